# baseline (speedup 1.0000x reference)
import functools

import jax
import jax.numpy as jnp
from jax import lax
from jax.experimental import pallas as pl
from jax.experimental.pallas import tpu as pltpu
from jax.experimental.pallas import tpu_sc as plsc

D = 128
NC = 2
NS = 16
NW = NC * NS
VL = 16
NV = D // VL


def kernel(user, item, user_table, item_table, W, b):
    B = user.shape[0]
    rpw = B // NW
    c = 64
    nchunk = rpw // c
    nbuf = 4
    user2 = user.astype(jnp.int32).reshape(NW, nchunk, c)
    item2 = item.astype(jnp.int32).reshape(NW, nchunk, c)
    w_flat = W.reshape(D)
    b_vec = jnp.broadcast_to(b.reshape(1), (VL,))

    mesh = plsc.VectorSubcoreMesh(
        core_axis_name="c", subcore_axis_name="s",
        num_cores=NC, num_subcores=NS)

    @functools.partial(
        pl.kernel,
        out_type=jax.ShapeDtypeStruct((B,), jnp.float32),
        mesh=mesh,
        compiler_params=pltpu.CompilerParams(needs_layout_passes=False),
        scratch_types=[
            pltpu.VMEM((nchunk, c), jnp.int32),
            pltpu.VMEM((nchunk, c), jnp.int32),
            pltpu.VMEM((nbuf, c, D), jnp.float32),
            pltpu.VMEM((nbuf, c, D), jnp.float32),
            pltpu.VMEM((D,), jnp.float32),
            pltpu.VMEM((VL,), jnp.float32),
            pltpu.VMEM((rpw,), jnp.float32),
            [pltpu.SemaphoreType.DMA] * nbuf,
            [pltpu.SemaphoreType.DMA] * nbuf,
        ],
    )
    def gmf(user_hbm, item_hbm, ut_hbm, it_hbm, w_hbm, b_hbm, out_hbm,
            uidx, iidx, urows, irows, w_v, b_v, out_v,
            usem, isem):
        cid = lax.axis_index("c")
        sid = lax.axis_index("s")
        wid = sid * NC + cid
        pltpu.sync_copy(user_hbm.at[wid], uidx)
        pltpu.sync_copy(item_hbm.at[wid], iidx)
        pltpu.sync_copy(w_hbm, w_v)
        pltpu.sync_copy(b_hbm, b_v)

        def start(j):
            buf = j % nbuf
            cu = pltpu.async_copy(ut_hbm.at[uidx.at[j]], urows.at[buf], usem[buf])
            ci = pltpu.async_copy(it_hbm.at[iidx.at[j]], irows.at[buf], isem[buf])
            return cu, ci

        wregs = [w_v[pl.ds(v * VL, VL)] for v in range(NV)]
        lanes = lax.broadcasted_iota(jnp.int32, (VL,), 0)
        lane0 = lanes == 0
        perms = [lanes ^ k for k in (8, 4, 2, 1)]

        def lane_sum(x):
            for p in perms:
                x = x + x[p]
            return x

        pending = [start(j) for j in range(nbuf - 1)]
        for j in range(nchunk):
            if j + nbuf - 1 < nchunk:
                pending.append(start(j + nbuf - 1))
            cu, ci = pending.pop(0)
            cu.wait()
            ci.wait()
            buf = j % nbuf
            u_ref = urows.at[buf]
            i_ref = irows.at[buf]

            @plsc.parallel_loop(0, c, unroll=8)
            def _row(r):
                acc = u_ref[r, pl.ds(0, VL)] * i_ref[r, pl.ds(0, VL)] * wregs[0]
                for v in range(1, NV):
                    acc = acc + (u_ref[r, pl.ds(v * VL, VL)]
                                 * i_ref[r, pl.ds(v * VL, VL)] * wregs[v])
                s = lane_sum(acc)
                pos = jnp.broadcast_to(j * c + r, (VL,)).astype(jnp.int32)
                plsc.store_scatter(out_v, [pos], s, mask=lane0)

        bb = b_v[...]
        for t in range(rpw // VL):
            x = out_v[pl.ds(t * VL, VL)]
            out_v[pl.ds(t * VL, VL)] = 1.0 / (1.0 + jnp.exp(-(x + bb)))

        pltpu.sync_copy(out_v, out_hbm.at[pl.ds(wid * rpw, rpw)])

    return gmf(user2, item2, user_table, item_table, w_flat, b_vec)


# speedup vs baseline: 1.0919x; 1.0919x over previous
import functools

import jax
import jax.numpy as jnp
from jax import lax
from jax.experimental import pallas as pl
from jax.experimental.pallas import tpu as pltpu
from jax.experimental.pallas import tpu_sc as plsc

D = 128
NC = 2
NS = 16
NW = NC * NS
VL = 16
NV = D // VL


def kernel(user, item, user_table, item_table, W, b):
    B = user.shape[0]
    rpw = B // NW
    c = 64
    nchunk = rpw // c
    nbuf = 4
    user2 = user.astype(jnp.int32).reshape(NW, nchunk, c)
    item2 = item.astype(jnp.int32).reshape(NW, nchunk, c)
    w_flat = W.reshape(D)
    b_vec = jnp.broadcast_to(b.reshape(1), (VL,))

    mesh = plsc.VectorSubcoreMesh(
        core_axis_name="c", subcore_axis_name="s",
        num_cores=NC, num_subcores=NS)

    @functools.partial(
        pl.kernel,
        out_type=jax.ShapeDtypeStruct((B,), jnp.float32),
        mesh=mesh,
        compiler_params=pltpu.CompilerParams(needs_layout_passes=False),
        scratch_types=[
            pltpu.VMEM((nchunk, c), jnp.int32),
            pltpu.VMEM((nchunk, c), jnp.int32),
            pltpu.VMEM((nbuf, c, D), jnp.float32),
            pltpu.VMEM((nbuf, c, D), jnp.float32),
            pltpu.VMEM((D,), jnp.float32),
            pltpu.VMEM((VL,), jnp.float32),
            pltpu.VMEM((rpw,), jnp.float32),
            [pltpu.SemaphoreType.DMA] * nbuf,
            [pltpu.SemaphoreType.DMA] * nbuf,
        ],
    )
    def gmf(user_hbm, item_hbm, ut_hbm, it_hbm, w_hbm, b_hbm, out_hbm,
            uidx, iidx, urows, irows, w_v, b_v, out_v,
            usem, isem):
        cid = lax.axis_index("c")
        sid = lax.axis_index("s")
        wid = sid * NC + cid
        pltpu.sync_copy(user_hbm.at[wid], uidx)
        pltpu.sync_copy(item_hbm.at[wid], iidx)
        pltpu.sync_copy(w_hbm, w_v)
        pltpu.sync_copy(b_hbm, b_v)

        def start(j):
            buf = j % nbuf
            cu = pltpu.async_copy(ut_hbm.at[uidx.at[j]], urows.at[buf], usem[buf])
            ci = pltpu.async_copy(it_hbm.at[iidx.at[j]], irows.at[buf], isem[buf])
            return cu, ci

        wregs = [w_v[pl.ds(v * VL, VL)] for v in range(NV)]
        lanes = lax.broadcasted_iota(jnp.int32, (VL,), 0)
        lane0 = lanes == 0
        perms = [lanes ^ k for k in (8, 4, 2, 1)]

        def lane_sum(x):
            for p in perms:
                x = x + x[p]
            return x

        pending = [start(j) for j in range(nbuf - 1)]
        for j in range(nchunk):
            if j + nbuf - 1 < nchunk:
                pending.append(start(j + nbuf - 1))
            cu, ci = pending.pop(0)
            cu.wait()
            ci.wait()
            buf = j % nbuf
            u_ref = urows.at[buf]
            i_ref = irows.at[buf]

            @plsc.parallel_loop(0, c, unroll=4)
            def _row(r):
                acc = u_ref[r, pl.ds(0, VL)] * i_ref[r, pl.ds(0, VL)] * wregs[0]
                for v in range(1, NV):
                    acc = acc + (u_ref[r, pl.ds(v * VL, VL)]
                                 * i_ref[r, pl.ds(v * VL, VL)] * wregs[v])
                s = lane_sum(acc)
                pos = jnp.broadcast_to(j * c + r, (VL,)).astype(jnp.int32)
                plsc.store_scatter(out_v, [pos], s, mask=lane0)

        bb = b_v[...]
        for t in range(rpw // VL):
            x = out_v[pl.ds(t * VL, VL)]
            out_v[pl.ds(t * VL, VL)] = 1.0 / (1.0 + jnp.exp(-(x + bb)))

        pltpu.sync_copy(out_v, out_hbm.at[pl.ds(wid * rpw, rpw)])

    return gmf(user2, item2, user_table, item_table, w_flat, b_vec)


# speedup vs baseline: 1.1453x; 1.0490x over previous
import functools

import jax
import jax.numpy as jnp
from jax import lax
from jax.experimental import pallas as pl
from jax.experimental.pallas import tpu as pltpu
from jax.experimental.pallas import tpu_sc as plsc

D = 128
NC = 2
NS = 16
NW = NC * NS
VL = 16
NV = D // VL


def kernel(user, item, user_table, item_table, W, b):
    B = user.shape[0]
    rpw = B // NW
    c = 128
    nchunk = rpw // c
    nbuf = 2
    user2 = user.astype(jnp.int32).reshape(NW, nchunk, c)
    item2 = item.astype(jnp.int32).reshape(NW, nchunk, c)
    w_flat = W.reshape(D)
    b_vec = jnp.broadcast_to(b.reshape(1), (VL,))

    mesh = plsc.VectorSubcoreMesh(
        core_axis_name="c", subcore_axis_name="s",
        num_cores=NC, num_subcores=NS)

    @functools.partial(
        pl.kernel,
        out_type=jax.ShapeDtypeStruct((B,), jnp.float32),
        mesh=mesh,
        compiler_params=pltpu.CompilerParams(needs_layout_passes=False),
        scratch_types=[
            pltpu.VMEM((nchunk, c), jnp.int32),
            pltpu.VMEM((nchunk, c), jnp.int32),
            pltpu.VMEM((nbuf, c, D), jnp.float32),
            pltpu.VMEM((nbuf, c, D), jnp.float32),
            pltpu.VMEM((D,), jnp.float32),
            pltpu.VMEM((VL,), jnp.float32),
            pltpu.VMEM((rpw,), jnp.float32),
            [pltpu.SemaphoreType.DMA] * nbuf,
            [pltpu.SemaphoreType.DMA] * nbuf,
        ],
    )
    def gmf(user_hbm, item_hbm, ut_hbm, it_hbm, w_hbm, b_hbm, out_hbm,
            uidx, iidx, urows, irows, w_v, b_v, out_v,
            usem, isem):
        cid = lax.axis_index("c")
        sid = lax.axis_index("s")
        wid = sid * NC + cid
        pltpu.sync_copy(user_hbm.at[wid], uidx)
        pltpu.sync_copy(item_hbm.at[wid], iidx)
        pltpu.sync_copy(w_hbm, w_v)
        pltpu.sync_copy(b_hbm, b_v)

        def start(j, slot):
            pltpu.async_copy(ut_hbm.at[uidx.at[j]], urows.at[slot], usem[slot])
            pltpu.async_copy(it_hbm.at[iidx.at[j]], irows.at[slot], isem[slot])

        def wait(slot):
            pltpu.make_async_copy(ut_hbm.at[uidx.at[0]], urows.at[slot],
                                  usem[slot]).wait()
            pltpu.make_async_copy(it_hbm.at[iidx.at[0]], irows.at[slot],
                                  isem[slot]).wait()

        wregs = [w_v[pl.ds(v * VL, VL)] for v in range(NV)]
        lanes = lax.broadcasted_iota(jnp.int32, (VL,), 0)
        lane0 = lanes == 0
        perms = [lanes ^ k for k in (8, 4, 2, 1)]

        for slot in range(nbuf):
            start(slot, slot)

        @pl.loop(0, nchunk // nbuf)
        def _group(g):
            for slot in range(nbuf):
                j = g * nbuf + slot
                wait(slot)
                u_ref = urows.at[slot]
                i_ref = irows.at[slot]

                @plsc.parallel_loop(0, c, unroll=4)
                def _row(r):
                    acc = u_ref[r, pl.ds(0, VL)] * i_ref[r, pl.ds(0, VL)] * wregs[0]
                    for v in range(1, NV):
                        acc = acc + (u_ref[r, pl.ds(v * VL, VL)]
                                     * i_ref[r, pl.ds(v * VL, VL)] * wregs[v])
                    for p in perms:
                        acc = acc + acc[p]
                    pos = jnp.broadcast_to(j * c + r, (VL,)).astype(jnp.int32)
                    plsc.store_scatter(out_v, [pos], acc, mask=lane0)

                @pl.when(j + nbuf < nchunk)
                def _():
                    start(j + nbuf, slot)

        bb = b_v[...]

        @pl.loop(0, rpw // VL)
        def _sig(t):
            x = out_v[pl.ds(t * VL, VL)]
            out_v[pl.ds(t * VL, VL)] = 1.0 / (1.0 + jnp.exp(-(x + bb)))

        pltpu.sync_copy(out_v, out_hbm.at[pl.ds(wid * rpw, rpw)])

    return gmf(user2, item2, user_table, item_table, w_flat, b_vec)
